# baseline (speedup 1.0000x reference)
import functools

import jax
import jax.numpy as jnp
from jax import lax
from jax.experimental import pallas as pl
from jax.experimental.pallas import tpu as pltpu
from jax.experimental.pallas import tpu_sc as plsc

B = 4096
H = 50
D = 64
DP = 128
VP = 100008
NC = 2
NS = 16
NW = NC * NS
BW = B // NW
L = 16


@functools.partial(
    pl.kernel,
    mesh=plsc.VectorSubcoreMesh(core_axis_name="c", subcore_axis_name="s"),
    out_type=jax.ShapeDtypeStruct((H, D, B), jnp.float32),
    scratch_types=[
        pltpu.VMEM((BW,), jnp.int32),
        pltpu.VMEM((BW,), jnp.int32),
        pltpu.VMEM((BW, DP), jnp.float32),
        pltpu.VMEM((BW, DP), jnp.float32),
        pltpu.VMEM((D, BW), jnp.float32),
        pltpu.VMEM((D, BW), jnp.float32),
        pltpu.SemaphoreType.DMA,
        pltpu.SemaphoreType.DMA,
        pltpu.SemaphoreType.DMA,
        pltpu.SemaphoreType.DMA,
        pltpu.SemaphoreType.DMA,
        pltpu.SemaphoreType.DMA,
    ],
    compiler_params=pltpu.CompilerParams(needs_layout_passes=False),
)
def _gather_t(idx_hbm, table_hbm, out_hbm, idx0, idx1, rows0, rows1,
              st0, st1, isem0, isem1, gsem0, gsem1, ssem0, ssem1):
    wid = lax.axis_index("s") * NC + lax.axis_index("c")
    b0 = wid * BW
    idxs = (idx0, idx1)
    rows = (rows0, rows1)
    stg = (st0, st1)
    isems = (isem0, isem1)
    gsems = (gsem0, gsem1)
    ssems = (ssem0, ssem1)

    def idxload(h):
        return pltpu.make_async_copy(
            idx_hbm.at[h, pl.ds(b0, BW)], idxs[h % 2], isems[h % 2])

    def gather(h):
        return pltpu.make_async_copy(
            table_hbm.at[idxs[h % 2]], rows[h % 2], gsems[h % 2])

    def store(h):
        return pltpu.make_async_copy(
            stg[h % 2], out_hbm.at[h, pl.ds(0, D), pl.ds(b0, BW)],
            ssems[h % 2])

    def transpose(h):
        src = rows[h % 2]
        dst = stg[h % 2]
        biota = lax.iota(jnp.int32, L)

        def dbody(d, _):
            dvec = jnp.full((L,), d, jnp.int32)
            for bg in range(BW // L):
                v = plsc.load_gather(src, [biota + bg * L, dvec])
                dst.at[d][pl.ds(bg * L, L)] = v
            return 0

        lax.fori_loop(0, D, dbody, 0)

    idxload(0).start()
    idxload(1).start()
    idxload(0).wait()
    gather(0).start()
    for h in range(H):
        if h + 1 < H:
            idxload(h + 1).wait()
            if h >= 1:
                store(h - 1).wait()
            gather(h + 1).start()
        gather(h).wait()
        if h + 2 < H:
            idxload(h + 2).start()
        transpose(h)
        store(h).start()
    store(H - 1).wait()


def kernel(user_ids, ID_embeddings):
    idx2 = user_ids.T.astype(jnp.int32)
    table = jnp.pad(ID_embeddings,
                    ((0, VP - ID_embeddings.shape[0]), (0, DP - D)))
    out = _gather_t(idx2, table)
    return jnp.transpose(out, (2, 0, 1))


# speedup vs baseline: 1.1831x; 1.1831x over previous
import functools

import jax
import jax.numpy as jnp
from jax import lax
from jax.experimental import pallas as pl
from jax.experimental.pallas import tpu as pltpu
from jax.experimental.pallas import tpu_sc as plsc

B = 4096
H = 50
D = 64
DP = 128
VP = 100008
NC = 2
NS = 16
NW = NC * NS
BW = B // NW
L = 16


@functools.partial(
    pl.kernel,
    mesh=plsc.VectorSubcoreMesh(core_axis_name="c", subcore_axis_name="s"),
    out_type=jax.ShapeDtypeStruct((H, D, B), jnp.float32),
    scratch_types=[
        pltpu.VMEM((BW,), jnp.int32),
        pltpu.VMEM((BW,), jnp.int32),
        pltpu.VMEM((BW, DP), jnp.float32),
        pltpu.VMEM((BW, DP), jnp.float32),
        pltpu.VMEM((D, BW + 1), jnp.float32),
        pltpu.VMEM((D, BW + 1), jnp.float32),
        pltpu.SemaphoreType.DMA,
        pltpu.SemaphoreType.DMA,
        pltpu.SemaphoreType.DMA,
        pltpu.SemaphoreType.DMA,
        pltpu.SemaphoreType.DMA,
        pltpu.SemaphoreType.DMA,
    ],
    compiler_params=pltpu.CompilerParams(needs_layout_passes=False),
)
def _gather_t(idx_hbm, table_hbm, out_hbm, idx0, idx1, rows0, rows1,
              st0, st1, isem0, isem1, gsem0, gsem1, ssem0, ssem1):
    wid = lax.axis_index("s") * NC + lax.axis_index("c")
    b0 = wid * BW
    idxs = (idx0, idx1)
    rows = (rows0, rows1)
    stg = (st0, st1)
    isems = (isem0, isem1)
    gsems = (gsem0, gsem1)
    ssems = (ssem0, ssem1)

    def idxload(h):
        return pltpu.make_async_copy(
            idx_hbm.at[h, pl.ds(b0, BW)], idxs[h % 2], isems[h % 2])

    def gather(h):
        return pltpu.make_async_copy(
            table_hbm.at[idxs[h % 2]], rows[h % 2], gsems[h % 2])

    def store(h):
        return pltpu.make_async_copy(
            stg[h % 2].at[pl.ds(0, D), pl.ds(0, BW)],
            out_hbm.at[h, pl.ds(0, D), pl.ds(b0, BW)],
            ssems[h % 2])

    def transpose(h):
        src = rows[h % 2]
        dst = stg[h % 2]
        diota = lax.iota(jnp.int32, L)

        def bbody(b, _):
            bvec = jnp.full((L,), b, jnp.int32)
            for dg in range(D // L):
                v = src.at[b][pl.ds(dg * L, L)]
                plsc.store_scatter(dst, [diota + dg * L, bvec], v)
            return 0

        lax.fori_loop(0, BW, bbody, 0)

    idxload(0).start()
    idxload(1).start()
    idxload(0).wait()
    gather(0).start()
    for h in range(H):
        if h + 1 < H:
            idxload(h + 1).wait()
            if h >= 1:
                store(h - 1).wait()
            gather(h + 1).start()
        gather(h).wait()
        if h + 2 < H:
            idxload(h + 2).start()
        transpose(h)
        store(h).start()
    store(H - 1).wait()


def kernel(user_ids, ID_embeddings):
    idx2 = user_ids.T.astype(jnp.int32)
    table = jnp.pad(ID_embeddings,
                    ((0, VP - ID_embeddings.shape[0]), (0, DP - D)))
    out = _gather_t(idx2, table)
    return jnp.transpose(out, (2, 0, 1))


# speedup vs baseline: 1.9970x; 1.6879x over previous
import functools

import jax
import jax.numpy as jnp
from jax import lax
from jax.experimental import pallas as pl
from jax.experimental.pallas import tpu as pltpu
from jax.experimental.pallas import tpu_sc as plsc

B = 4096
H = 50
TOTAL = B * H
D = 64
DP = 128
VP = 100008
NC = 2
NS = 16
NW = NC * NS
B_PER_W = B // NW
PER_W = TOTAL // NW
BCH = 4
CHUNK = BCH * H
NCHUNK = PER_W // CHUNK
NBUF = 4


@functools.partial(
    pl.kernel,
    mesh=plsc.VectorSubcoreMesh(core_axis_name="c", subcore_axis_name="s"),
    out_type=jax.ShapeDtypeStruct((B, H, DP), jnp.float32),
    scratch_types=(
        [pltpu.VMEM((CHUNK,), jnp.int32) for _ in range(NBUF)]
        + [pltpu.VMEM((CHUNK, DP), jnp.float32) for _ in range(NBUF)]
        + [pltpu.SemaphoreType.DMA] * (3 * NBUF)
    ),
)
def _gather_rows(idx_hbm, table_hbm, out_hbm, *scratch):
    idxs = scratch[:NBUF]
    bufs = scratch[NBUF:2 * NBUF]
    isems = scratch[2 * NBUF:3 * NBUF]
    gsems = scratch[3 * NBUF:4 * NBUF]
    ssems = scratch[4 * NBUF:5 * NBUF]
    wid = lax.axis_index("s") * NC + lax.axis_index("c")
    base = wid * PER_W
    bbase = wid * B_PER_W

    def idxload(g):
        return pltpu.make_async_copy(
            idx_hbm.at[pl.ds(base + g * CHUNK, CHUNK)],
            idxs[g % NBUF], isems[g % NBUF])

    def gather(g):
        return pltpu.make_async_copy(
            table_hbm.at[idxs[g % NBUF]], bufs[g % NBUF], gsems[g % NBUF])

    def stores(g):
        return [
            pltpu.make_async_copy(
                bufs[g % NBUF].at[pl.ds(i * H, H)],
                out_hbm.at[bbase + g * BCH + i],
                ssems[g % NBUF])
            for i in range(BCH)
        ]

    for g in range(NBUF):
        idxload(g).start()
    idxload(0).wait()
    gather(0).start()
    idxload(1).wait()
    gather(1).start()
    for g in range(NCHUNK):
        if g >= 2:
            for s in stores(g - 2):
                s.wait()
        if g + 2 < NCHUNK:
            idxload(g + 2).wait()
            gather(g + 2).start()
        gather(g).wait()
        if g + NBUF < NCHUNK:
            idxload(g + NBUF).start()
        for s in stores(g):
            s.start()
    for s in stores(NCHUNK - 2):
        s.wait()
    for s in stores(NCHUNK - 1):
        s.wait()


def kernel(user_ids, ID_embeddings):
    idx = user_ids.reshape(-1).astype(jnp.int32)
    table = jnp.pad(ID_embeddings,
                    ((0, VP - ID_embeddings.shape[0]), (0, DP - D)))
    out = _gather_rows(idx, table)
    return out[:, :, :D]
